# baseline (speedup 1.0000x reference)
import functools

import jax
import jax.numpy as jnp
from jax import lax
from jax.experimental import pallas as pl
from jax.experimental.pallas import tpu as pltpu
from jax.experimental.pallas import tpu_sc as plsc

N = 10000
E = 320000
D = 128
NH = 5120
NPAD = 2 * NH
ACCR = 5248
TRASH = 5184
ZRPT = ACCR // 16
ORPT = NH // 16
CH = 128
NCH = 160
EPT = NCH * CH
EPAD = 16 * EPT
PADROW = N

_MESH = plsc.VectorSubcoreMesh(core_axis_name="c", subcore_axis_name="s")


@functools.partial(
    pl.kernel,
    out_type=jax.ShapeDtypeStruct((NPAD, D), jnp.float32),
    mesh=_MESH,
    scratch_types=[
        pltpu.VMEM_SHARED((ACCR, D), jnp.float32),
        pltpu.VMEM((NCH, CH), jnp.int32),
        pltpu.VMEM((NCH, CH), jnp.int32),
        pltpu.VMEM((2, CH), jnp.int32),
        pltpu.VMEM((2, CH), jnp.int32),
        pltpu.VMEM((CH, D), jnp.float32),
        pltpu.VMEM((CH, D), jnp.float32),
        pltpu.SemaphoreType.DMA,
        pltpu.SemaphoreType.DMA,
        pltpu.SemaphoreType.DMA,
        pltpu.SemaphoreType.DMA,
    ],
)
def _sc_pass(table_h, si, di, zeros, out,
             acc, sidx, didx, sidx1, didx1, rows0, rows1,
             gsem0, gsem1, ssem0, ssem1):
    c = lax.axis_index("c")
    s = lax.axis_index("s")
    base = c * NH

    pltpu.sync_copy(zeros.at[pl.ds(s * ZRPT, ZRPT)],
                    acc.at[pl.ds(s * ZRPT, ZRPT)])
    pltpu.sync_copy(si.at[s], sidx)
    pltpu.sync_copy(di.at[s], didx)
    plsc.subcore_barrier()

    rows = (rows0, rows1)
    gsem = (gsem0, gsem1)
    ssem = (ssem0, ssem1)

    def fill_sidx(p, j):
        for k in range(CH // 16):
            sl = pl.ds(k * 16, 16)
            sidx1[p, sl] = sidx[j, sl]

    def fill_didx(p, j):
        for k in range(CH // 16):
            sl = pl.ds(k * 16, 16)
            v = didx[j, sl] - base
            ok = (v >= 0) & (v < NH)
            didx1[p, sl] = jnp.where(ok, v, TRASH)

    def gather(p, j):
        del j
        return pltpu.async_copy(table_h.at[sidx1.at[p]], rows[p], gsem[p])

    def scatter(p):
        return pltpu.async_copy(rows[p], acc.at[didx1.at[p]], ssem[p],
                                add=True)

    for p in (0, 1):
        fill_sidx(p, p)
        fill_didx(p, p)
        gather(p, p)

    @pl.loop(0, NCH // 2 - 1)
    def _(jj):
        for p in (0, 1):
            j = 2 * jj + p
            pltpu.make_async_copy(table_h, rows[p], gsem[p]).wait()
            d = scatter(p)
            fill_sidx(p, j + 2)
            d.wait()
            fill_didx(p, j + 2)
            gather(p, j + 2)

    for p in (0, 1):
        pltpu.make_async_copy(table_h, rows[p], gsem[p]).wait()
        scatter(p).wait()

    plsc.subcore_barrier()
    pltpu.sync_copy(acc.at[pl.ds(s * ORPT, ORPT)],
                    out.at[pl.ds(base + s * ORPT, ORPT)])


_RB = 512
_GRID = NPAD // _RB


def _tc_pre_body(h_ref, deg_ref, o_ref):
    norm = lax.rsqrt(jnp.maximum(deg_ref[:, :1], 1.0))
    o_ref[...] = h_ref[...] * norm


def _tc_pre(h_pad, out_deg):
    return pl.pallas_call(
        _tc_pre_body,
        grid=(_GRID,),
        in_specs=[pl.BlockSpec((_RB, D), lambda i: (i, 0)),
                  pl.BlockSpec((_RB, D), lambda i: (i, 0))],
        out_specs=pl.BlockSpec((_RB, D), lambda i: (i, 0)),
        out_shape=jax.ShapeDtypeStruct((NPAD, D), jnp.float32),
    )(h_pad, out_deg)


def _tc_mid_body(a_ref, ind_ref, outd_ref, w_ref, bias_ref, o_ref):
    agg = a_ref[...] * lax.rsqrt(jnp.maximum(ind_ref[:, :1], 1.0))
    y = jnp.dot(agg, w_ref[...], preferred_element_type=jnp.float32,
                precision=lax.Precision.HIGHEST)
    y = jnp.maximum(y + bias_ref[...], 0.0)
    o_ref[...] = y * lax.rsqrt(jnp.maximum(outd_ref[:, :1], 1.0))


def _tc_mid(acc, in_deg, out_deg, W, bias):
    return pl.pallas_call(
        _tc_mid_body,
        grid=(_GRID,),
        in_specs=[pl.BlockSpec((_RB, D), lambda i: (i, 0)),
                  pl.BlockSpec((_RB, D), lambda i: (i, 0)),
                  pl.BlockSpec((_RB, D), lambda i: (i, 0)),
                  pl.BlockSpec((D, D), lambda i: (0, 0)),
                  pl.BlockSpec((1, D), lambda i: (0, 0))],
        out_specs=pl.BlockSpec((_RB, D), lambda i: (i, 0)),
        out_shape=jax.ShapeDtypeStruct((NPAD, D), jnp.float32),
    )(acc, in_deg, out_deg, W, bias)


def _tc_last_body(a_ref, ind_ref, w_ref, bias_ref, o_ref):
    agg = a_ref[...] * lax.rsqrt(jnp.maximum(ind_ref[:, :1], 1.0))
    o_ref[...] = jnp.dot(agg, w_ref[...], preferred_element_type=jnp.float32,
                         precision=lax.Precision.HIGHEST) + bias_ref[...]


def _tc_last(acc, in_deg, W, bias):
    return pl.pallas_call(
        _tc_last_body,
        grid=(_GRID,),
        in_specs=[pl.BlockSpec((_RB, D), lambda i: (i, 0)),
                  pl.BlockSpec((_RB, D), lambda i: (i, 0)),
                  pl.BlockSpec((D, D), lambda i: (0, 0)),
                  pl.BlockSpec((1, D), lambda i: (0, 0))],
        out_specs=pl.BlockSpec((_RB, D), lambda i: (i, 0)),
        out_shape=jax.ShapeDtypeStruct((N, D), jnp.float32),
    )(acc, in_deg, W, bias)


def kernel(h, e, edge_index, W0, b0, W1, b1, W2, b2):
    src = edge_index[0]
    dst = edge_index[1]
    pad_idx = jnp.full((EPAD - E,), PADROW, dtype=jnp.int32)
    src_t = jnp.concatenate([src, pad_idx]).reshape(16, NCH, CH)
    dst_t = jnp.concatenate([dst, pad_idx]).reshape(16, NCH, CH)
    h_pad = jnp.pad(h, ((0, NPAD - N), (0, 0)))
    ones2 = jnp.ones((NPAD, D), jnp.float32)
    zeroacc = jnp.zeros((ACCR, D), jnp.float32)

    zidx = jnp.zeros((16, NCH, CH), jnp.int32)
    out_deg = _sc_pass(ones2, zidx, src_t, zeroacc)
    in_deg = _sc_pass(ones2, zidx, dst_t, zeroacc)
    h2 = _tc_pre(h_pad, out_deg)

    out = None
    for (W, b, last) in ((W0, b0, False), (W1, b1, False), (W2, b2, True)):
        agg = _sc_pass(h2, src_t, dst_t, zeroacc)
        if last:
            out = _tc_last(agg, in_deg, W, b.reshape(1, D))
        else:
            h2 = _tc_mid(agg, in_deg, out_deg, W, b.reshape(1, D))
    return (out, e)


# speedup vs baseline: 17.3724x; 17.3724x over previous
import functools

import jax
import jax.numpy as jnp
from jax import lax
from jax.experimental import pallas as pl
from jax.experimental.pallas import tpu as pltpu
from jax.experimental.pallas import tpu_sc as plsc

N = 10000
E = 320000
D = 128
NH = 5120
NPAD = 2 * NH
ACCR = 5248
TRASH = 5184
ZRPT = ACCR // 16
ORPT = NH // 16
CH = 128
NCH = 160
EPT = NCH * CH
EPAD = 16 * EPT
PADROW = N

_MESH = plsc.VectorSubcoreMesh(core_axis_name="c", subcore_axis_name="s")


@functools.partial(
    pl.kernel,
    out_type=jax.ShapeDtypeStruct((NPAD, D), jnp.float32),
    mesh=_MESH,
    scratch_types=[
        pltpu.VMEM_SHARED((ACCR, D), jnp.float32),
        pltpu.VMEM((NCH, CH), jnp.int32),
        pltpu.VMEM((NCH, CH), jnp.int32),
        pltpu.VMEM((2, CH), jnp.int32),
        pltpu.VMEM((2, CH), jnp.int32),
        pltpu.VMEM((CH, D), jnp.float32),
        pltpu.VMEM((CH, D), jnp.float32),
        pltpu.SemaphoreType.DMA,
        pltpu.SemaphoreType.DMA,
        pltpu.SemaphoreType.DMA,
        pltpu.SemaphoreType.DMA,
    ],
)
def _sc_pass(table_h, si, di, zeros, out,
             acc, sidx, didx, sidx1, didx1, rows0, rows1,
             gsem0, gsem1, ssem0, ssem1):
    c = lax.axis_index("c")
    s = lax.axis_index("s")
    base = c * NH

    pltpu.sync_copy(zeros.at[pl.ds(s * ZRPT, ZRPT)],
                    acc.at[pl.ds(s * ZRPT, ZRPT)])
    pltpu.sync_copy(si.at[s], sidx)
    pltpu.sync_copy(di.at[s], didx)
    plsc.subcore_barrier()

    rows = (rows0, rows1)
    gsem = (gsem0, gsem1)
    ssem = (ssem0, ssem1)

    def fill_sidx(p, j):
        for k in range(CH // 16):
            sl = pl.ds(k * 16, 16)
            sidx1[p, sl] = sidx[j, sl]

    def fill_didx(p, j):
        for k in range(CH // 16):
            sl = pl.ds(k * 16, 16)
            v = didx[j, sl] - base
            ok = (v >= 0) & (v < NH)
            didx1[p, sl] = jnp.where(ok, v, TRASH)

    def gather(p, j):
        del j
        return pltpu.async_copy(table_h.at[sidx1.at[p]], rows[p], gsem[p])

    def scatter(p):
        return pltpu.async_copy(rows[p], acc.at[didx1.at[p]], ssem[p],
                                add=True)

    for p in (0, 1):
        fill_sidx(p, p)
        fill_didx(p, p)
        gather(p, p)

    @pl.loop(0, NCH // 2 - 1)
    def _(jj):
        for p in (0, 1):
            j = 2 * jj + p
            pltpu.make_async_copy(table_h, rows[p], gsem[p]).wait()
            d = scatter(p)
            fill_sidx(p, j + 2)
            d.wait()
            fill_didx(p, j + 2)
            gather(p, j + 2)

    for p in (0, 1):
        pltpu.make_async_copy(table_h, rows[p], gsem[p]).wait()
        scatter(p).wait()

    plsc.subcore_barrier()
    pltpu.sync_copy(acc.at[pl.ds(s * ORPT, ORPT)],
                    out.at[pl.ds(base + s * ORPT, ORPT)])


_RB = 512
_GRID = NPAD // _RB


def _tc_pre_body(h_ref, deg_ref, o_ref):
    norm = lax.rsqrt(jnp.maximum(deg_ref[:, :1], 1.0))
    o_ref[...] = h_ref[...] * norm


def _tc_pre(h_pad, out_deg):
    return pl.pallas_call(
        _tc_pre_body,
        grid=(_GRID,),
        in_specs=[pl.BlockSpec((_RB, D), lambda i: (i, 0)),
                  pl.BlockSpec((_RB, D), lambda i: (i, 0))],
        out_specs=pl.BlockSpec((_RB, D), lambda i: (i, 0)),
        out_shape=jax.ShapeDtypeStruct((NPAD, D), jnp.float32),
    )(h_pad, out_deg)


def _tc_mid_body(a_ref, ind_ref, outd_ref, w_ref, bias_ref, o_ref):
    agg = a_ref[...] * lax.rsqrt(jnp.maximum(ind_ref[:, :1], 1.0))
    y = jnp.dot(agg, w_ref[...], preferred_element_type=jnp.float32,
                precision=lax.Precision.HIGHEST)
    y = jnp.maximum(y + bias_ref[...], 0.0)
    o_ref[...] = y * lax.rsqrt(jnp.maximum(outd_ref[:, :1], 1.0))


def _tc_mid(acc, in_deg, out_deg, W, bias):
    return pl.pallas_call(
        _tc_mid_body,
        grid=(_GRID,),
        in_specs=[pl.BlockSpec((_RB, D), lambda i: (i, 0)),
                  pl.BlockSpec((_RB, D), lambda i: (i, 0)),
                  pl.BlockSpec((_RB, D), lambda i: (i, 0)),
                  pl.BlockSpec((D, D), lambda i: (0, 0)),
                  pl.BlockSpec((1, D), lambda i: (0, 0))],
        out_specs=pl.BlockSpec((_RB, D), lambda i: (i, 0)),
        out_shape=jax.ShapeDtypeStruct((NPAD, D), jnp.float32),
    )(acc, in_deg, out_deg, W, bias)


def _tc_last_body(a_ref, ind_ref, w_ref, bias_ref, o_ref):
    agg = a_ref[...] * lax.rsqrt(jnp.maximum(ind_ref[:, :1], 1.0))
    o_ref[...] = jnp.dot(agg, w_ref[...], preferred_element_type=jnp.float32,
                         precision=lax.Precision.HIGHEST) + bias_ref[...]


def _tc_last(acc, in_deg, W, bias):
    return pl.pallas_call(
        _tc_last_body,
        grid=(_GRID,),
        in_specs=[pl.BlockSpec((_RB, D), lambda i: (i, 0)),
                  pl.BlockSpec((_RB, D), lambda i: (i, 0)),
                  pl.BlockSpec((D, D), lambda i: (0, 0)),
                  pl.BlockSpec((1, D), lambda i: (0, 0))],
        out_specs=pl.BlockSpec((_RB, D), lambda i: (i, 0)),
        out_shape=jax.ShapeDtypeStruct((N, D), jnp.float32),
    )(acc, in_deg, W, bias)


def kernel(h, e, edge_index, W0, b0, W1, b1, W2, b2):
    src = edge_index[0]
    dst = edge_index[1]
    pad_idx = jnp.full((EPAD - E,), PADROW, dtype=jnp.int32)
    src_t = jnp.concatenate([src, pad_idx]).reshape(16, NCH, CH)
    dst_t = jnp.concatenate([dst, pad_idx]).reshape(16, NCH, CH)
    h_pad = jnp.pad(h, ((0, NPAD - N), (0, 0)))
    ones2 = jnp.ones((NPAD, D), jnp.float32)
    zeroacc = jnp.zeros((ACCR, D), jnp.float32)

    seq_idx = (jnp.arange(EPAD, dtype=jnp.int32) % NPAD).reshape(16, NCH, CH)
    out_deg = _sc_pass(ones2, seq_idx, src_t, zeroacc)
    in_deg = _sc_pass(ones2, seq_idx, dst_t, zeroacc)
    h2 = _tc_pre(h_pad, out_deg)

    out = None
    for (W, b, last) in ((W0, b0, False), (W1, b1, False), (W2, b2, True)):
        agg = _sc_pass(h2, src_t, dst_t, zeroacc)
        if last:
            out = _tc_last(agg, in_deg, W, b.reshape(1, D))
        else:
            h2 = _tc_mid(agg, in_deg, out_deg, W, b.reshape(1, D))
    return (out, e)
